# baseline (speedup 1.0000x reference)
import functools

import jax
import jax.numpy as jnp
from jax import lax
from jax.experimental import pallas as pl
from jax.experimental.pallas import tpu as pltpu
from jax.experimental.pallas import tpu_sc as plsc

_NUM_CORES = 2
_NUM_SUBCORES = 16
_NW = _NUM_CORES * _NUM_SUBCORES
_BB = 128


def _gather_kernel(b, t, d, dp):
    n_per_w = _BB * t
    mesh = plsc.VectorSubcoreMesh(
        core_axis_name="c",
        subcore_axis_name="s",
        num_cores=_NUM_CORES,
        num_subcores=_NUM_SUBCORES,
    )

    @functools.partial(
        pl.kernel,
        out_type=jax.ShapeDtypeStruct((t, d, b), jnp.float32),
        mesh=mesh,
        scratch_types=[
            pltpu.VMEM((n_per_w,), jnp.int32),
            pltpu.VMEM((t, _BB), jnp.int32),
            pltpu.VMEM((2, _BB, dp), jnp.float32),
            pltpu.VMEM((2, d, _BB), jnp.float32),
            pltpu.SemaphoreType.DMA,
            pltpu.SemaphoreType.DMA,
            pltpu.SemaphoreType.DMA,
            pltpu.SemaphoreType.DMA,
        ],
        compiler_params=pltpu.CompilerParams(
            use_tc_tiling_on_sc=True, needs_layout_passes=False
        ),
    )
    def k(idx_hbm, table_hbm, out_hbm, idx_v, idx_t, rows_v, obuf, sg0, sg1,
          sw0, sw1):
        wid = lax.axis_index("s") * _NUM_CORES + lax.axis_index("c")
        b0 = wid * _BB
        pltpu.sync_copy(idx_hbm.at[pl.ds(b0 * t, n_per_w)], idx_v)
        sg = (sg0, sg1)
        sw = (sw0, sw1)
        iota16 = jnp.arange(16, dtype=jnp.int32)
        i_t = iota16 * t

        def idx_t_body(tt, carry):
            for jb in range(_BB // 16):
                vec = i_t + (jb * 16 * t + tt)
                idx_t[tt, pl.ds(jb * 16, 16)] = plsc.load_gather(idx_v, [vec])
            return carry

        lax.fori_loop(0, t, idx_t_body, 0, unroll=2)

        def start_gather(tt, k_):
            pltpu.async_copy(
                table_hbm.at[idx_t.at[tt]], rows_v.at[k_], sg[k_]
            )

        def wait_gather(k_):
            pltpu.make_async_copy(
                table_hbm.at[idx_t.at[0]], rows_v.at[k_], sg[k_]
            ).wait()

        def start_write(tt, k_):
            pltpu.async_copy(
                obuf.at[k_], out_hbm.at[tt, :, pl.ds(b0, _BB)], sw[k_]
            )

        def wait_write(k_):
            pltpu.make_async_copy(
                obuf.at[k_], out_hbm.at[0, :, pl.ds(b0, _BB)], sw[k_]
            ).wait()

        def transpose(k_):
            rows2 = rows_v.at[k_]
            ob = obuf.at[k_]

            def e_body(eb, carry):
                for ei in range(8):
                    e = eb * 8 + ei
                    col = jnp.full((16,), 0, jnp.int32) + e
                    for jb in range(_BB // 16):
                        row = iota16 + (jb * 16)
                        ob[e, pl.ds(jb * 16, 16)] = plsc.load_gather(
                            rows2, [row, col]
                        )
                return carry

            lax.fori_loop(0, d // 8, e_body, 0)

        def step(tt, k_, do_wait_write, next_t):
            wait_gather(k_)
            if do_wait_write:
                wait_write(k_)
            transpose(k_)
            start_write(tt, k_)
            if next_t is not None:
                start_gather(next_t, k_)

        start_gather(0, 0)
        start_gather(1, 1)
        step(0, 0, False, 2)
        step(1, 1, False, 3)

        def body(t2, carry):
            tt = 2 * t2
            step(tt, 0, True, tt + 2)
            step(tt + 1, 1, True, tt + 3)
            return carry

        lax.fori_loop(1, t // 2 - 1, body, 0)

        step(t - 2, 0, True, None)
        step(t - 1, 1, True, None)
        wait_write(0)
        wait_write(1)

    return k


def kernel(idx, table):
    b, t = idx.shape
    v, d = table.shape
    table_p = jnp.pad(table, ((0, 0), (0, 128 - d)))
    out = _gather_kernel(b, t, d, 128)(idx.reshape(b * t), table_p)
    return jnp.transpose(out, (2, 0, 1))


# speedup vs baseline: 1.7499x; 1.7499x over previous
import functools

import jax
import jax.numpy as jnp
from jax import lax
from jax.experimental import pallas as pl
from jax.experimental.pallas import tpu as pltpu
from jax.experimental.pallas import tpu_sc as plsc

_NUM_CORES = 2
_NUM_SUBCORES = 16
_NW = _NUM_CORES * _NUM_SUBCORES
_CHUNK = 200
_GROUP = 2


def _gather_kernel(n, dp):
    n_per_w = n // _NW
    n_chunks = n_per_w // _CHUNK
    n_groups = n_chunks // _GROUP
    mesh = plsc.VectorSubcoreMesh(
        core_axis_name="c",
        subcore_axis_name="s",
        num_cores=_NUM_CORES,
        num_subcores=_NUM_SUBCORES,
    )

    @functools.partial(
        pl.kernel,
        out_type=jax.ShapeDtypeStruct((n, dp), jnp.float32),
        mesh=mesh,
        scratch_types=[
            pltpu.VMEM((n_per_w,), jnp.int32),
            pltpu.VMEM((_GROUP * 2, _CHUNK, dp), jnp.float32),
            pltpu.SemaphoreType.DMA,
            pltpu.SemaphoreType.DMA,
            pltpu.SemaphoreType.DMA,
            pltpu.SemaphoreType.DMA,
        ],
        compiler_params=pltpu.CompilerParams(use_tc_tiling_on_sc=True),
    )
    def k(idx_hbm, table_hbm, out_hbm, idx_v, rows_v, sg0, sg1, sw0, sw1):
        wid = lax.axis_index("s") * _NUM_CORES + lax.axis_index("c")
        base = wid * n_per_w
        pltpu.sync_copy(idx_hbm.at[pl.ds(base, n_per_w)], idx_v)
        sg = (sg0, sg1)
        sw = (sw0, sw1)

        def start_gathers(grp, p):
            for q in range(_GROUP):
                off = (grp * _GROUP + q) * _CHUNK
                pltpu.async_copy(
                    table_hbm.at[idx_v.at[pl.ds(off, _CHUNK)]],
                    rows_v.at[_GROUP * p + q],
                    sg[p],
                )

        def wait_gathers(p):
            for q in range(_GROUP):
                pltpu.make_async_copy(
                    table_hbm.at[idx_v.at[pl.ds(0, _CHUNK)]],
                    rows_v.at[_GROUP * p + q],
                    sg[p],
                ).wait()

        def start_writes(grp, p):
            for q in range(_GROUP):
                off = (grp * _GROUP + q) * _CHUNK
                pltpu.async_copy(
                    rows_v.at[_GROUP * p + q],
                    out_hbm.at[pl.ds(base + off, _CHUNK)],
                    sw[p],
                )

        def wait_writes(p):
            for q in range(_GROUP):
                pltpu.make_async_copy(
                    rows_v.at[_GROUP * p + q],
                    out_hbm.at[pl.ds(base, _CHUNK)],
                    sw[p],
                ).wait()

        def run_group(grp, p, wait_prev_writes, start_next):
            wait_gathers(p)
            if wait_prev_writes:
                wait_writes(1 - p)
            if start_next:
                start_gathers(grp + 1, 1 - p)
            start_writes(grp, p)

        start_gathers(0, 0)
        run_group(0, 0, False, True)
        run_group(1, 1, True, True)

        def body(jj, carry):
            run_group(2 * jj, 0, True, True)
            run_group(2 * jj + 1, 1, True, True)
            return carry

        lax.fori_loop(1, n_groups // 2 - 1, body, 0)

        run_group(n_groups - 2, 0, True, True)
        run_group(n_groups - 1, 1, True, False)
        wait_writes(1)

    return k


def kernel(idx, table):
    b, t = idx.shape
    v, d = table.shape
    n = b * t
    table_p = jnp.pad(table, ((0, 0), (0, 128 - d)))
    out = _gather_kernel(n, 128)(idx.reshape(n), table_p)
    return out[:, :d].reshape(b, t, d)


# speedup vs baseline: 1.8531x; 1.0590x over previous
import functools

import jax
import jax.numpy as jnp
from jax import lax
from jax.experimental import pallas as pl
from jax.experimental.pallas import tpu as pltpu
from jax.experimental.pallas import tpu_sc as plsc

_NUM_CORES = 2
_NUM_SUBCORES = 16
_NW = _NUM_CORES * _NUM_SUBCORES
_CHUNK = 200
_GROUP = 2


def _gather_kernel(n, dp):
    n_per_w = n // _NW
    n_chunks = n_per_w // _CHUNK
    n_groups = n_chunks // _GROUP
    mesh = plsc.VectorSubcoreMesh(
        core_axis_name="c",
        subcore_axis_name="s",
        num_cores=_NUM_CORES,
        num_subcores=_NUM_SUBCORES,
    )

    @functools.partial(
        pl.kernel,
        out_type=jax.ShapeDtypeStruct((n, dp), jnp.float32),
        mesh=mesh,
        scratch_types=[
            pltpu.VMEM((n_per_w,), jnp.int32),
            pltpu.VMEM((_GROUP * 2, _CHUNK, dp), jnp.float32),
            pltpu.SemaphoreType.DMA,
            pltpu.SemaphoreType.DMA,
            pltpu.SemaphoreType.DMA,
            pltpu.SemaphoreType.DMA,
        ],
        compiler_params=pltpu.CompilerParams(use_tc_tiling_on_sc=True),
    )
    def k(idx_hbm, table_hbm, out_hbm, idx_v, rows_v, sg0, sg1, sw0, sw1):
        wid = lax.axis_index("s") * _NUM_CORES + lax.axis_index("c")
        base = wid * n_per_w
        pltpu.sync_copy(idx_hbm.at[pl.ds(base, n_per_w)], idx_v)
        sg = (sg0, sg1)
        sw = (sw0, sw1)

        def start_gathers(grp, p):
            for q in range(_GROUP):
                off = (grp * _GROUP + q) * _CHUNK
                pltpu.async_copy(
                    table_hbm.at[idx_v.at[pl.ds(off, _CHUNK)]],
                    rows_v.at[_GROUP * p + q],
                    sg[p],
                )

        def wait_gathers(p):
            for q in range(_GROUP):
                pltpu.make_async_copy(
                    table_hbm.at[idx_v.at[pl.ds(0, _CHUNK)]],
                    rows_v.at[_GROUP * p + q],
                    sg[p],
                ).wait()

        def start_writes(grp, p):
            for q in range(_GROUP):
                off = (grp * _GROUP + q) * _CHUNK
                pltpu.async_copy(
                    rows_v.at[_GROUP * p + q],
                    out_hbm.at[pl.ds(base + off, _CHUNK)],
                    sw[p],
                )

        def wait_writes(p):
            for q in range(_GROUP):
                pltpu.make_async_copy(
                    rows_v.at[_GROUP * p + q],
                    out_hbm.at[pl.ds(base, _CHUNK)],
                    sw[p],
                ).wait()

        def run_group(grp, p, wait_prev_writes, start_next):
            wait_gathers(p)
            if wait_prev_writes:
                wait_writes(1 - p)
            if start_next:
                start_gathers(grp + 1, 1 - p)
            start_writes(grp, p)

        start_gathers(0, 0)
        run_group(0, 0, False, True)
        run_group(1, 1, True, True)

        def body(jj, carry):
            run_group(2 * jj, 0, True, True)
            run_group(2 * jj + 1, 1, True, True)
            return carry

        lax.fori_loop(1, n_groups // 2 - 1, body, 0)

        run_group(n_groups - 2, 0, True, True)
        run_group(n_groups - 1, 1, True, False)
        wait_writes(1)

    return k


def _pad_transpose(v, d):
    blk = 2048

    def body(tt_ref, out_ref):
        out_ref[:, :d] = tt_ref[...].T
        out_ref[:, d:] = jnp.zeros((blk, 128 - d), jnp.float32)

    return pl.pallas_call(
        body,
        grid=((v + blk - 1) // blk,),
        in_specs=[pl.BlockSpec((d, blk), lambda j: (0, j))],
        out_specs=pl.BlockSpec((blk, 128), lambda j: (j, 0)),
        out_shape=jax.ShapeDtypeStruct((v, 128), jnp.float32),
    )


def kernel(idx, table):
    b, t = idx.shape
    v, d = table.shape
    n = b * t
    table_p = _pad_transpose(v, d)(table.T)
    out = _gather_kernel(n, 128)(idx.reshape(n), table_p)
    return out[:, :d].reshape(b, t, d)


# speedup vs baseline: 2.3441x; 1.2650x over previous
import functools

import jax
import jax.numpy as jnp
from jax import lax
from jax.experimental import pallas as pl
from jax.experimental.pallas import tpu as pltpu
from jax.experimental.pallas import tpu_sc as plsc

_NUM_CORES = 2
_NUM_SUBCORES = 16
_NW = _NUM_CORES * _NUM_SUBCORES
_CHUNK = 200
_GROUP = 2


def _gather_kernel(n, dp):
    n_per_w = n // _NW
    n_chunks = n_per_w // _CHUNK
    n_groups = n_chunks // _GROUP
    mesh = plsc.VectorSubcoreMesh(
        core_axis_name="c",
        subcore_axis_name="s",
        num_cores=_NUM_CORES,
        num_subcores=_NUM_SUBCORES,
    )

    @functools.partial(
        pl.kernel,
        out_type=jax.ShapeDtypeStruct((n, dp), jnp.float32),
        mesh=mesh,
        scratch_types=[
            pltpu.VMEM((n_per_w,), jnp.int32),
            pltpu.VMEM((_GROUP * 2, _CHUNK, dp), jnp.float32),
            pltpu.SemaphoreType.DMA,
            pltpu.SemaphoreType.DMA,
            pltpu.SemaphoreType.DMA,
            pltpu.SemaphoreType.DMA,
        ],
        compiler_params=pltpu.CompilerParams(use_tc_tiling_on_sc=True),
    )
    def k(idx_hbm, table_hbm, out_hbm, idx_v, rows_v, sg0, sg1, sw0, sw1):
        wid = lax.axis_index("s") * _NUM_CORES + lax.axis_index("c")
        base = wid * n_per_w
        pltpu.sync_copy(idx_hbm.at[pl.ds(base, n_per_w)], idx_v)
        sg = (sg0, sg1)
        sw = (sw0, sw1)

        def start_gathers(grp, p):
            for q in range(_GROUP):
                off = (grp * _GROUP + q) * _CHUNK
                pltpu.async_copy(
                    table_hbm.at[idx_v.at[pl.ds(off, _CHUNK)]],
                    rows_v.at[_GROUP * p + q],
                    sg[p],
                )

        def wait_gathers(p):
            for q in range(_GROUP):
                pltpu.make_async_copy(
                    table_hbm.at[idx_v.at[pl.ds(0, _CHUNK)]],
                    rows_v.at[_GROUP * p + q],
                    sg[p],
                ).wait()

        def start_writes(grp, p):
            for q in range(_GROUP):
                off = (grp * _GROUP + q) * _CHUNK
                pltpu.async_copy(
                    rows_v.at[_GROUP * p + q],
                    out_hbm.at[pl.ds(base + off, _CHUNK)],
                    sw[p],
                )

        def wait_writes(p):
            for q in range(_GROUP):
                pltpu.make_async_copy(
                    rows_v.at[_GROUP * p + q],
                    out_hbm.at[pl.ds(base, _CHUNK)],
                    sw[p],
                ).wait()

        def run_group(grp, p, wait_prev_writes, start_next):
            wait_gathers(p)
            if wait_prev_writes:
                wait_writes(1 - p)
            if start_next:
                start_gathers(grp + 1, 1 - p)
            start_writes(grp, p)

        start_gathers(0, 0)
        run_group(0, 0, False, True)
        run_group(1, 1, True, True)

        def body(jj, carry):
            run_group(2 * jj, 0, True, True)
            run_group(2 * jj + 1, 1, True, True)
            return carry

        lax.fori_loop(1, n_groups // 2 - 1, body, 0)

        run_group(n_groups - 2, 0, True, True)
        run_group(n_groups - 1, 1, True, False)
        wait_writes(1)

    return k


def _pad_transpose(v, d):
    blk = 8192

    def body(tt_ref, out_ref):
        out_ref[:, :d] = tt_ref[...].T
        out_ref[:, d:] = jnp.zeros((blk, 128 - d), jnp.float32)

    return pl.pallas_call(
        body,
        grid=((v + blk - 1) // blk,),
        in_specs=[pl.BlockSpec((d, blk), lambda j: (0, j))],
        out_specs=pl.BlockSpec((blk, 128), lambda j: (j, 0)),
        out_shape=jax.ShapeDtypeStruct((v, 128), jnp.float32),
    )


def kernel(idx, table):
    b, t = idx.shape
    v, d = table.shape
    n = b * t
    table_p = _pad_transpose(v, d)(table.T)
    out = _gather_kernel(n, 128)(idx.reshape(n), table_p)
    return out[:, :d].reshape(b, t, d)


# speedup vs baseline: 2.4003x; 1.0240x over previous
import functools

import jax
import jax.numpy as jnp
from jax import lax
from jax.experimental import pallas as pl
from jax.experimental.pallas import tpu as pltpu
from jax.experimental.pallas import tpu_sc as plsc

_NUM_CORES = 2
_NUM_SUBCORES = 16
_NW = _NUM_CORES * _NUM_SUBCORES
_CHUNK = 200
_GROUP = 2


def _gather_kernel(n, dp):
    n_per_w = n // _NW
    n_chunks = n_per_w // _CHUNK
    n_groups = n_chunks // _GROUP
    mesh = plsc.VectorSubcoreMesh(
        core_axis_name="c",
        subcore_axis_name="s",
        num_cores=_NUM_CORES,
        num_subcores=_NUM_SUBCORES,
    )

    @functools.partial(
        pl.kernel,
        out_type=jax.ShapeDtypeStruct((n, dp), jnp.float32),
        mesh=mesh,
        scratch_types=[
            pltpu.VMEM((n_per_w,), jnp.int32),
            pltpu.VMEM((_GROUP * 2, _CHUNK, dp), jnp.float32),
            pltpu.SemaphoreType.DMA,
            pltpu.SemaphoreType.DMA,
            pltpu.SemaphoreType.DMA,
            pltpu.SemaphoreType.DMA,
        ],
        compiler_params=pltpu.CompilerParams(use_tc_tiling_on_sc=True),
    )
    def k(idx_hbm, table_hbm, out_hbm, idx_v, rows_v, sg0, sg1, sw0, sw1):
        wid = lax.axis_index("s") * _NUM_CORES + lax.axis_index("c")
        base = wid * n_per_w
        pltpu.sync_copy(idx_hbm.at[pl.ds(base, n_per_w)], idx_v)
        sg = (sg0, sg1)
        sw = (sw0, sw1)

        def start_gathers(grp, p):
            for q in range(_GROUP):
                off = (grp * _GROUP + q) * _CHUNK
                pltpu.async_copy(
                    table_hbm.at[idx_v.at[pl.ds(off, _CHUNK)]],
                    rows_v.at[_GROUP * p + q],
                    sg[p],
                )

        def wait_gathers(p):
            for q in range(_GROUP):
                pltpu.make_async_copy(
                    table_hbm.at[idx_v.at[pl.ds(0, _CHUNK)]],
                    rows_v.at[_GROUP * p + q],
                    sg[p],
                ).wait()

        def start_writes(grp, p):
            for q in range(_GROUP):
                off = (grp * _GROUP + q) * _CHUNK
                pltpu.async_copy(
                    rows_v.at[_GROUP * p + q],
                    out_hbm.at[pl.ds(base + off, _CHUNK)],
                    sw[p],
                )

        def wait_writes(p):
            for q in range(_GROUP):
                pltpu.make_async_copy(
                    rows_v.at[_GROUP * p + q],
                    out_hbm.at[pl.ds(base, _CHUNK)],
                    sw[p],
                ).wait()

        def run_group(grp, p, wait_prev_writes, start_next):
            wait_gathers(p)
            if wait_prev_writes:
                wait_writes(1 - p)
            if start_next:
                start_gathers(grp + 1, 1 - p)
            start_writes(grp, p)

        start_gathers(0, 0)
        run_group(0, 0, False, True)
        run_group(1, 1, True, True)

        def body(jj, carry):
            run_group(2 * jj, 0, True, True)
            run_group(2 * jj + 1, 1, True, True)
            return carry

        lax.fori_loop(1, n_groups // 2 - 1, body, 0)

        run_group(n_groups - 2, 0, True, True)
        run_group(n_groups - 1, 1, True, False)
        wait_writes(1)

    return k


def _pad_transpose(v, d):
    blk = 16384

    def body(tt_ref, out_ref):
        out_ref[:, :d] = tt_ref[...].T
        out_ref[:, d:] = jnp.zeros((blk, 128 - d), jnp.float32)

    return pl.pallas_call(
        body,
        grid=((v + blk - 1) // blk,),
        in_specs=[pl.BlockSpec((d, blk), lambda j: (0, j))],
        out_specs=pl.BlockSpec((blk, 128), lambda j: (j, 0)),
        out_shape=jax.ShapeDtypeStruct((v, 128), jnp.float32),
    )


def kernel(idx, table):
    b, t = idx.shape
    v, d = table.shape
    n = b * t
    table_p = _pad_transpose(v, d)(table.T)
    out = _gather_kernel(n, 128)(idx.reshape(n), table_p)
    return out[:, :d].reshape(b, t, d)


# speedup vs baseline: 2.4304x; 1.0125x over previous
import functools

import jax
import jax.numpy as jnp
from jax import lax
from jax.experimental import pallas as pl
from jax.experimental.pallas import tpu as pltpu
from jax.experimental.pallas import tpu_sc as plsc

_NUM_CORES = 2
_NUM_SUBCORES = 16
_NW = _NUM_CORES * _NUM_SUBCORES
_CHUNK = 200
_GROUP = 2


def _gather_kernel(n, dp):
    n_per_w = n // _NW
    n_chunks = n_per_w // _CHUNK
    n_groups = n_chunks // _GROUP
    mesh = plsc.VectorSubcoreMesh(
        core_axis_name="c",
        subcore_axis_name="s",
        num_cores=_NUM_CORES,
        num_subcores=_NUM_SUBCORES,
    )

    @functools.partial(
        pl.kernel,
        out_type=jax.ShapeDtypeStruct((n, dp), jnp.float32),
        mesh=mesh,
        scratch_types=[
            pltpu.VMEM((n_per_w,), jnp.int32),
            pltpu.VMEM((_GROUP * 2, _CHUNK, dp), jnp.float32),
            pltpu.SemaphoreType.DMA,
            pltpu.SemaphoreType.DMA,
            pltpu.SemaphoreType.DMA,
            pltpu.SemaphoreType.DMA,
        ],
        compiler_params=pltpu.CompilerParams(use_tc_tiling_on_sc=True),
    )
    def k(idx_hbm, table_hbm, out_hbm, idx_v, rows_v, sg0, sg1, sw0, sw1):
        wid = lax.axis_index("s") * _NUM_CORES + lax.axis_index("c")
        base = wid * n_per_w
        pltpu.sync_copy(idx_hbm.at[pl.ds(base, n_per_w)], idx_v)
        sg = (sg0, sg1)
        sw = (sw0, sw1)

        def start_gathers(grp, p):
            for q in range(_GROUP):
                off = (grp * _GROUP + q) * _CHUNK
                pltpu.async_copy(
                    table_hbm.at[idx_v.at[pl.ds(off, _CHUNK)]],
                    rows_v.at[_GROUP * p + q],
                    sg[p],
                )

        def wait_gathers(p):
            for q in range(_GROUP):
                pltpu.make_async_copy(
                    table_hbm.at[idx_v.at[pl.ds(0, _CHUNK)]],
                    rows_v.at[_GROUP * p + q],
                    sg[p],
                ).wait()

        def start_writes(grp, p):
            for q in range(_GROUP):
                off = (grp * _GROUP + q) * _CHUNK
                pltpu.async_copy(
                    rows_v.at[_GROUP * p + q],
                    out_hbm.at[pl.ds(base + off, _CHUNK)],
                    sw[p],
                )

        def wait_writes(p):
            for q in range(_GROUP):
                pltpu.make_async_copy(
                    rows_v.at[_GROUP * p + q],
                    out_hbm.at[pl.ds(base, _CHUNK)],
                    sw[p],
                ).wait()

        def run_group(grp, p, wait_prev_writes, start_next):
            wait_gathers(p)
            if wait_prev_writes:
                wait_writes(1 - p)
            if start_next:
                start_gathers(grp + 1, 1 - p)
            start_writes(grp, p)

        start_gathers(0, 0)
        run_group(0, 0, False, True)
        run_group(1, 1, True, True)

        def body(jj, carry):
            run_group(2 * jj, 0, True, True)
            run_group(2 * jj + 1, 1, True, True)
            return carry

        lax.fori_loop(1, n_groups // 2 - 1, body, 0)

        run_group(n_groups - 2, 0, True, True)
        run_group(n_groups - 1, 1, True, False)
        wait_writes(1)

    return k


def _pad_transpose(v, d):
    blk = 32768

    def body(tt_ref, out_ref):
        out_ref[:, :d] = tt_ref[...].T

    return pl.pallas_call(
        body,
        grid=((v + blk - 1) // blk,),
        in_specs=[pl.BlockSpec((d, blk), lambda j: (0, j))],
        out_specs=pl.BlockSpec((blk, 128), lambda j: (j, 0)),
        out_shape=jax.ShapeDtypeStruct((v, 128), jnp.float32),
    )


def kernel(idx, table):
    b, t = idx.shape
    v, d = table.shape
    n = b * t
    table_p = _pad_transpose(v, d)(table.T)
    out = _gather_kernel(n, 128)(idx.reshape(n), table_p)
    return out[:, :d].reshape(b, t, d)
